# baseline (speedup 1.0000x reference)
import jax
import jax.numpy as jnp
from jax import lax
from jax.experimental import pallas as pl
from jax.experimental.pallas import tpu as pltpu
from jax.experimental.pallas import tpu_sc as plsc

_NC, _NS = 2, 16
_NW = _NC * _NS
_CHUNK = 128
_LANES = 16
_NBUF = 2
_NIDX = 4


def _round_up(a: int, b: int) -> int:
    return (a + b - 1) // b * b


def _make_sc_scatter(n_pad: int, per_tile: int, d: int, with_count: bool):
    rows_pt = n_pad // _NS
    full_chunks = per_tile // _CHUNK
    tail = per_tile - full_chunks * _CHUNK
    assert full_chunks >= _NIDX and per_tile % 8 == 0 and tail % 8 == 0

    def body(table, src_h, dst_h, *rest):
        if with_count:
            z2, acc_out, cnt_out, *more = rest
        else:
            z2, acc_out, *more = rest
        it = iter(more)
        sbuf = [next(it) for _ in range(_NIDX)]
        dbuf = [next(it) for _ in range(_NIDX)]
        rows_v = [next(it) for _ in range(_NBUF)]
        if tail:
            tsrc, tdst, trows = next(it), next(it), next(it)
        if with_count:
            ones_v = next(it)
            cbuf = next(it)
        acc_sh = next(it)
        if with_count:
            cnt_sh = next(it)
        gsems = [next(it) for _ in range(_NBUF)]
        isems = [next(it) for _ in range(_NIDX)]
        jsems = [next(it) for _ in range(_NIDX)]

        c = lax.axis_index("c")
        s = lax.axis_index("s")
        w = c * _NS + s
        base = w * per_tile
        row0 = s * rows_pt
        rows = pl.ds(row0, rows_pt)

        for q in range(_NIDX):
            pltpu.async_copy(src_h.at[pl.ds(base + q * _CHUNK, _CHUNK)],
                             sbuf[q], isems[q])
            pltpu.async_copy(dst_h.at[pl.ds(base + q * _CHUNK, _CHUNK)],
                             dbuf[q], jsems[q])

        pltpu.sync_copy(z2.at[rows], acc_sh.at[rows])
        if with_count:
            for j in range(rows_pt // _LANES):
                cbuf[pl.ds(j * _LANES, _LANES)] = jnp.zeros(
                    (_LANES,), jnp.float32)
            pltpu.sync_copy(cbuf, cnt_sh.at[rows])
            for j in range(_CHUNK // _LANES):
                ones_v[pl.ds(j * _LANES, _LANES)] = jnp.ones(
                    (_LANES,), jnp.float32)
        plsc.subcore_barrier()

        for b in range(_NBUF):
            pltpu.make_async_copy(
                src_h.at[pl.ds(base + b * _CHUNK, _CHUNK)],
                sbuf[b], isems[b]).wait()
            pltpu.async_copy(table.at[sbuf[b]], rows_v[b], gsems[b])

        def issue_next_gather(idx, u, b):
            ng = idx + _NBUF
            q2 = (u + _NBUF) % _NIDX
            pltpu.make_async_copy(
                src_h.at[pl.ds(base + ng * _CHUNK, _CHUNK)],
                sbuf[q2], isems[q2]).wait()
            pltpu.async_copy(table.at[sbuf[q2]], rows_v[b], gsems[b])

        def chunk_step(idx, u, in_loop):
            b = u % _NBUF
            pltpu.make_async_copy(table.at[sbuf[u]],
                                  rows_v[b], gsems[b]).wait()
            pltpu.make_async_copy(
                dst_h.at[pl.ds(base + idx * _CHUNK, _CHUNK)],
                dbuf[u], jsems[u]).wait()
            pltpu.sync_copy(rows_v[b], acc_sh.at[dbuf[u]], add=True)
            if with_count:
                pltpu.sync_copy(ones_v, cnt_sh.at[dbuf[u]], add=True)
            if in_loop:
                @pl.when(idx + _NIDX < full_chunks)
                def _():
                    nb = idx + _NIDX
                    pltpu.async_copy(
                        src_h.at[pl.ds(base + nb * _CHUNK, _CHUNK)],
                        sbuf[u], isems[u])
                    pltpu.async_copy(
                        dst_h.at[pl.ds(base + nb * _CHUNK, _CHUNK)],
                        dbuf[u], jsems[u])

                @pl.when(idx + _NBUF < full_chunks)
                def _():
                    issue_next_gather(idx, u, b)
            else:
                if idx + _NBUF < full_chunks:
                    issue_next_gather(idx, u, b)

        n_quads = full_chunks // _NIDX
        if n_quads * _NIDX == full_chunks:
            n_quads -= 1

        def quad(blk, carry):
            for u in range(_NIDX):
                chunk_step(blk * _NIDX + u, u, True)
            return carry

        lax.fori_loop(0, n_quads, quad, 0)
        for idx in range(n_quads * _NIDX, full_chunks):
            chunk_step(idx, idx % _NIDX, False)

        if tail:
            toff = full_chunks * _CHUNK
            pltpu.sync_copy(src_h.at[pl.ds(base + toff, tail)], tsrc)
            pltpu.sync_copy(dst_h.at[pl.ds(base + toff, tail)], tdst)
            pltpu.async_copy(table.at[tsrc], trows, gsems[0]).wait()
            pltpu.sync_copy(trows, acc_sh.at[tdst], add=True)
            if with_count:
                pltpu.sync_copy(ones_v.at[pl.ds(0, tail)],
                                cnt_sh.at[tdst], add=True)

        plsc.subcore_barrier()
        pltpu.sync_copy(acc_sh.at[rows], acc_out.at[c, rows])
        if with_count:
            pltpu.sync_copy(cnt_sh.at[rows], cbuf)
            pltpu.sync_copy(cbuf,
                            cnt_out.at[pl.ds(c * n_pad + row0, rows_pt)])

    out_type = [jax.ShapeDtypeStruct((_NC, n_pad, d), jnp.float32)]
    scratch = [pltpu.VMEM((_CHUNK,), jnp.int32) for _ in range(_NIDX)]
    scratch += [pltpu.VMEM((_CHUNK,), jnp.int32) for _ in range(_NIDX)]
    scratch += [pltpu.VMEM((_CHUNK, d), jnp.float32)
                for _ in range(_NBUF)]
    if tail:
        scratch += [pltpu.VMEM((tail,), jnp.int32),
                    pltpu.VMEM((tail,), jnp.int32),
                    pltpu.VMEM((tail, d), jnp.float32)]
    if with_count:
        out_type.append(jax.ShapeDtypeStruct((_NC * n_pad,), jnp.float32))
        scratch.append(pltpu.VMEM((_CHUNK,), jnp.float32))
        scratch.append(pltpu.VMEM((rows_pt,), jnp.float32))
    scratch.append(pltpu.VMEM_SHARED((n_pad, d), jnp.float32))
    if with_count:
        scratch.append(pltpu.VMEM_SHARED((n_pad,), jnp.float32))
    scratch += [pltpu.SemaphoreType.DMA
                for _ in range(_NBUF + 2 * _NIDX)]

    mesh = plsc.VectorSubcoreMesh(core_axis_name="c", subcore_axis_name="s",
                                  num_cores=_NC, num_subcores=_NS)
    return pl.kernel(body, out_type=out_type, mesh=mesh,
                     scratch_types=scratch)


def _matmul_body(x_ref, w_ref, o_ref):
    o_ref[...] = jnp.dot(x_ref[...], w_ref[...],
                         preferred_element_type=jnp.float32)


def _mid_body(p0, p1, r, b, w, o):
    h = (p0[0] + p1[0]) * r[...] + b[...]
    h = jnp.where(h >= 0.0, h, 0.01 * h)
    o[...] = jnp.dot(h, w[...], preferred_element_type=jnp.float32)


def _fin_body(p0, p1, r, b, o):
    o[...] = (p0[0] + p1[0]) * r[...] + b[...]


def _tc_call(body, n, n_pad, d, bn, w_shape=None, out_d=None):
    out_d = out_d or d
    grid = (n // bn,)
    in_specs = [pl.BlockSpec((1, bn, d), lambda i: (0, i, 0)),
                pl.BlockSpec((1, bn, d), lambda i: (1, i, 0)),
                pl.BlockSpec((bn, 1), lambda i: (i, 0)),
                pl.BlockSpec((1, out_d), lambda i: (0, 0))]
    if w_shape is not None:
        in_specs.append(pl.BlockSpec(w_shape, lambda i: (0, 0)))
    return pl.pallas_call(
        body, grid=grid, in_specs=in_specs,
        out_specs=pl.BlockSpec((bn, out_d), lambda i: (i, 0)),
        out_shape=jax.ShapeDtypeStruct((n, out_d), jnp.float32))


def kernel(x, edge_index, W1, b1, W2, b2):
    n, d_in = x.shape
    d_h = W1.shape[0]
    d_out = W2.shape[0]
    e = edge_index.shape[1]

    n_pad = _round_up(n + 1, _NS * _LANES)
    src = edge_index[0].astype(jnp.int32)
    dst = edge_index[1].astype(jnp.int32)
    if e % (_NW * 8) != 0:
        e_pad = _round_up(e, _NW * 8)
        pad = e_pad - e
        ar = jnp.arange(pad, dtype=jnp.int32)
        src = jnp.concatenate([src, ar % n])
        dst = jnp.concatenate([dst, n + ar % (n_pad - n)])
    else:
        e_pad = e
    per_tile = e_pad // _NW

    z2 = jnp.zeros((n_pad, d_h), jnp.float32)

    bn = 2000
    mm1 = pl.pallas_call(
        _matmul_body, grid=(n // bn,),
        in_specs=[pl.BlockSpec((bn, d_in), lambda i: (i, 0)),
                  pl.BlockSpec((d_in, d_h), lambda i: (0, 0))],
        out_specs=pl.BlockSpec((bn, d_h), lambda i: (i, 0)),
        out_shape=jax.ShapeDtypeStruct((n, d_h), jnp.float32))
    g1 = mm1(x, W1.T)

    scat1 = _make_sc_scatter(n_pad, per_tile, d_h, with_count=True)
    parts1, counts = scat1(g1, src, dst, z2)
    counts = counts.reshape(_NC, n_pad)
    rinv = (1.0 / jnp.maximum(counts[0] + counts[1], 1.0))[:, None]

    mid = _tc_call(_mid_body, n, n_pad, d_h, bn,
                   w_shape=(d_h, d_out), out_d=d_out)
    g2 = mid(parts1, parts1, rinv, b1.reshape(1, d_h), W2.T)

    scat2 = _make_sc_scatter(n_pad, per_tile, d_out, with_count=False)
    (parts2,) = scat2(g2, src, dst, z2)

    fin = _tc_call(_fin_body, n, n_pad, d_out, bn)
    out = fin(parts2, parts2, rinv, b2.reshape(1, d_out))
    return out


# speedup vs baseline: 1.0604x; 1.0604x over previous
import jax
import jax.numpy as jnp
from jax import lax
from jax.experimental import pallas as pl
from jax.experimental.pallas import tpu as pltpu
from jax.experimental.pallas import tpu_sc as plsc

_NC, _NS = 2, 16
_NW = _NC * _NS
_CHUNK = 128
_LANES = 16
_NBUF = 2
_NIDX = 4


def _round_up(a: int, b: int) -> int:
    return (a + b - 1) // b * b


def _make_sc_scatter(n_pad: int, n_edges: int, d: int, with_count: bool):
    rows_pt = n_pad // _NS
    total_chunks = n_edges // _CHUNK
    tail = n_edges - total_chunks * _CHUNK
    full_chunks = total_chunks // _NW
    extra = total_chunks - full_chunks * _NW
    assert full_chunks >= _NIDX and tail % 8 == 0

    def body(table, eix, *rest):
        if with_count:
            z2, acc_out, cnt_out, *more = rest
        else:
            z2, acc_out, *more = rest
        it = iter(more)
        ebuf = [next(it) for _ in range(_NIDX)]
        rows_v = [next(it) for _ in range(_NBUF)]
        xbuf, trows = ebuf[0], rows_v[0]
        if with_count:
            ones_v = next(it)
            cbuf = next(it)
        acc_sh = next(it)
        if with_count:
            cnt_sh = next(it)
        gsems = [next(it) for _ in range(_NBUF)]
        isems = [next(it) for _ in range(_NIDX)]

        c = lax.axis_index("c")
        s = lax.axis_index("s")
        w = c * _NS + s
        row0 = s * rows_pt
        rows = pl.ds(row0, rows_pt)

        def col(j):
            return (j * _NW + w) * _CHUNK

        for q in range(_NIDX):
            pltpu.async_copy(eix.at[:, pl.ds(col(q), _CHUNK)],
                             ebuf[q], isems[q])

        pltpu.sync_copy(z2.at[rows], acc_sh.at[rows])
        if with_count:
            for j in range(rows_pt // _LANES):
                cbuf[pl.ds(j * _LANES, _LANES)] = jnp.zeros(
                    (_LANES,), jnp.float32)
            pltpu.sync_copy(cbuf, cnt_sh.at[rows])
            for j in range(_CHUNK // _LANES):
                ones_v[pl.ds(j * _LANES, _LANES)] = jnp.ones(
                    (_LANES,), jnp.float32)
        plsc.subcore_barrier()

        for b in range(_NBUF):
            pltpu.make_async_copy(eix.at[:, pl.ds(col(b), _CHUNK)],
                                  ebuf[b], isems[b]).wait()
            pltpu.async_copy(table.at[ebuf[b].at[0]], rows_v[b], gsems[b])

        def issue_next_gather(idx, u, b):
            ng = idx + _NBUF
            q2 = (u + _NBUF) % _NIDX
            pltpu.make_async_copy(eix.at[:, pl.ds(col(ng), _CHUNK)],
                                  ebuf[q2], isems[q2]).wait()
            pltpu.async_copy(table.at[ebuf[q2].at[0]], rows_v[b], gsems[b])

        def chunk_step(idx, u, in_loop):
            b = u % _NBUF
            pltpu.make_async_copy(table.at[ebuf[u].at[0]],
                                  rows_v[b], gsems[b]).wait()
            pltpu.sync_copy(rows_v[b], acc_sh.at[ebuf[u].at[1]], add=True)
            if with_count:
                pltpu.sync_copy(ones_v, cnt_sh.at[ebuf[u].at[1]], add=True)
            if in_loop:
                @pl.when(idx + _NIDX < full_chunks)
                def _():
                    pltpu.async_copy(
                        eix.at[:, pl.ds(col(idx + _NIDX), _CHUNK)],
                        ebuf[u], isems[u])

                @pl.when(idx + _NBUF < full_chunks)
                def _():
                    issue_next_gather(idx, u, b)
            else:
                if idx + _NBUF < full_chunks:
                    issue_next_gather(idx, u, b)

        n_quads = full_chunks // _NIDX
        if n_quads * _NIDX == full_chunks:
            n_quads -= 1

        def quad(blk, carry):
            for u in range(_NIDX):
                chunk_step(blk * _NIDX + u, u, True)
            return carry

        lax.fori_loop(0, n_quads, quad, 0)
        for idx in range(n_quads * _NIDX, full_chunks):
            chunk_step(idx, idx % _NIDX, False)

        if extra:
            @pl.when(w < extra)
            def _():
                pltpu.sync_copy(eix.at[:, pl.ds(col(full_chunks), _CHUNK)],
                                xbuf)
                pltpu.async_copy(table.at[xbuf.at[0]], trows,
                                 gsems[0]).wait()
                pltpu.sync_copy(trows, acc_sh.at[xbuf.at[1]], add=True)
                if with_count:
                    pltpu.sync_copy(ones_v, cnt_sh.at[xbuf.at[1]],
                                    add=True)
        if tail:
            @pl.when(s == 0)
            def _():
                toff = total_chunks * _CHUNK
                pltpu.sync_copy(eix.at[:, pl.ds(toff, tail)],
                                xbuf.at[:, pl.ds(0, tail)])
                pltpu.async_copy(table.at[xbuf.at[0, pl.ds(0, tail)]],
                                 trows.at[pl.ds(0, tail)],
                                 gsems[0]).wait()
                pltpu.sync_copy(trows.at[pl.ds(0, tail)],
                                acc_sh.at[xbuf.at[1, pl.ds(0, tail)]],
                                add=True)
                if with_count:
                    pltpu.sync_copy(ones_v.at[pl.ds(0, tail)],
                                    cnt_sh.at[xbuf.at[1, pl.ds(0, tail)]],
                                    add=True)

        plsc.subcore_barrier()
        pltpu.sync_copy(acc_sh.at[rows], acc_out.at[c, rows])
        if with_count:
            pltpu.sync_copy(cnt_sh.at[rows], cbuf)
            pltpu.sync_copy(cbuf,
                            cnt_out.at[pl.ds(c * n_pad + row0, rows_pt)])

    out_type = [jax.ShapeDtypeStruct((_NC, n_pad, d), jnp.float32)]
    scratch = [pltpu.VMEM((2, _CHUNK), jnp.int32) for _ in range(_NIDX)]
    scratch += [pltpu.VMEM((_CHUNK, d), jnp.float32)
                for _ in range(_NBUF)]
    if with_count:
        out_type.append(jax.ShapeDtypeStruct((_NC * n_pad,), jnp.float32))
        scratch.append(pltpu.VMEM((_CHUNK,), jnp.float32))
        scratch.append(pltpu.VMEM((rows_pt,), jnp.float32))
    scratch.append(pltpu.VMEM_SHARED((n_pad, d), jnp.float32))
    if with_count:
        scratch.append(pltpu.VMEM_SHARED((n_pad,), jnp.float32))
    scratch += [pltpu.SemaphoreType.DMA
                for _ in range(_NBUF + _NIDX)]

    mesh = plsc.VectorSubcoreMesh(core_axis_name="c", subcore_axis_name="s",
                                  num_cores=_NC, num_subcores=_NS)
    return pl.kernel(body, out_type=out_type, mesh=mesh,
                     scratch_types=scratch)


def _matmul_body(x_ref, w_ref, o_ref):
    o_ref[...] = jnp.dot(x_ref[...], w_ref[...],
                         preferred_element_type=jnp.float32)


def _mid_body(p0, p1, r, b, w, o):
    h = (p0[0] + p1[0]) * r[...] + b[...]
    h = jnp.where(h >= 0.0, h, 0.01 * h)
    o[...] = jnp.dot(h, w[...], preferred_element_type=jnp.float32)


def _fin_body(p0, p1, r, b, o):
    o[...] = (p0[0] + p1[0]) * r[...] + b[...]


def _tc_call(body, n, n_pad, d, bn, w_shape=None, out_d=None):
    out_d = out_d or d
    grid = (n // bn,)
    in_specs = [pl.BlockSpec((1, bn, d), lambda i: (0, i, 0)),
                pl.BlockSpec((1, bn, d), lambda i: (1, i, 0)),
                pl.BlockSpec((bn, 1), lambda i: (i, 0)),
                pl.BlockSpec((1, out_d), lambda i: (0, 0))]
    if w_shape is not None:
        in_specs.append(pl.BlockSpec(w_shape, lambda i: (0, 0)))
    return pl.pallas_call(
        body, grid=grid, in_specs=in_specs,
        out_specs=pl.BlockSpec((bn, out_d), lambda i: (i, 0)),
        out_shape=jax.ShapeDtypeStruct((n, out_d), jnp.float32))


def kernel(x, edge_index, W1, b1, W2, b2):
    n, d_in = x.shape
    d_h = W1.shape[0]
    d_out = W2.shape[0]
    e = edge_index.shape[1]

    n_pad = _round_up(n + 1, _NS * _LANES)
    eix = edge_index.astype(jnp.int32)

    z2 = jnp.zeros((n_pad, d_h), jnp.float32)

    bn = 2000
    mm1 = pl.pallas_call(
        _matmul_body, grid=(n // bn,),
        in_specs=[pl.BlockSpec((bn, d_in), lambda i: (i, 0)),
                  pl.BlockSpec((d_in, d_h), lambda i: (0, 0))],
        out_specs=pl.BlockSpec((bn, d_h), lambda i: (i, 0)),
        out_shape=jax.ShapeDtypeStruct((n, d_h), jnp.float32))
    g1 = mm1(x, W1.T)

    scat1 = _make_sc_scatter(n_pad, e, d_h, with_count=True)
    parts1, counts = scat1(g1, eix, z2)
    counts = counts.reshape(_NC, n_pad)
    rinv = (1.0 / jnp.maximum(counts[0] + counts[1], 1.0))[:, None]

    mid = _tc_call(_mid_body, n, n_pad, d_h, bn,
                   w_shape=(d_h, d_out), out_d=d_out)
    g2 = mid(parts1, parts1, rinv, b1.reshape(1, d_h), W2.T)

    scat2 = _make_sc_scatter(n_pad, e, d_out, with_count=False)
    (parts2,) = scat2(g2, eix, z2)

    fin = _tc_call(_fin_body, n, n_pad, d_out, bn)
    out = fin(parts2, parts2, rinv, b2.reshape(1, d_out))
    return out
